# baseline (speedup 1.0000x reference)
import functools

import numpy as np
import jax
import jax.numpy as jnp
from jax import lax
from jax.experimental import pallas as pl
from jax.experimental.pallas import tpu as pltpu
from jax.experimental.pallas import tpu_sc as plsc

_VOCAB = 100000
_HPAD = 100096
_NC, _NS, _L = 2, 16, 16
_NW = _NC * _NS
_ROWS, _COLS = 16384, 200
_N = _ROWS * _COLS
_PER_W = _N // _NW
_CHUNK = 6400
_NCHUNK = _PER_W // _CHUNK
_GROUPS = _CHUNK // _L
_RUNROLL = 1

_CH_ROWS = _CHUNK // _COLS
_ROWS_W = _ROWS // _NW


def _sc_hist_body(rec_hbm, out_hbm, hist, buf0, buf1, sem0, sem1):
    wid = lax.axis_index("s") * _NC + lax.axis_index("c")
    base_row = wid * _ROWS_W
    bufs, sems = (buf0, buf1), (sem0, sem1)

    pending = pltpu.async_copy(
        rec_hbm.at[pl.ds(base_row, _CH_ROWS), :], bufs[0], sems[0]
    )

    zeros = jnp.zeros((_L,), jnp.int32)

    def _zero(i, carry):
        hist[pl.ds(i * _L, _L)] = zeros
        return carry

    lax.fori_loop(0, _HPAD // _L, _zero, 0, unroll=8)

    ones = jnp.ones((_L,), jnp.int32)
    _himask = jnp.arange(_L, dtype=jnp.int32) >= _L - _COLS % _L
    for c in range(_NCHUNK):
        cur = pending
        if c + 1 < _NCHUNK:
            pending = pltpu.async_copy(
                rec_hbm.at[pl.ds(base_row + (c + 1) * _CH_ROWS, _CH_ROWS), :],
                bufs[(c + 1) % 2],
                sems[(c + 1) % 2],
            )
        cur.wait()
        buf = bufs[c % 2]

        def _rows(r0, carry2):
            for r in range(_RUNROLL):
                idxs = [buf[r0 + r, pl.ds(g * _L, _L)] for g in range(12)]
                tail = buf[r0 + r, pl.ds(_COLS - _L, _L)]
                for idx in idxs:
                    plsc.addupdate_scatter(hist, [idx], ones)
                plsc.addupdate_scatter(hist, [tail], ones, mask=_himask)
            return carry2

        lax.fori_loop(0, _CH_ROWS // _RUNROLL,
                      lambda r, c2: _rows(r * _RUNROLL, c2), 0)

    pltpu.sync_copy(hist, out_hbm.at[wid])


@functools.cache
def _get_sc_hist():
    mesh = plsc.VectorSubcoreMesh(
        core_axis_name="c", subcore_axis_name="s", num_cores=_NC, num_subcores=_NS
    )
    return pl.kernel(
        _sc_hist_body,
        mesh=mesh,
        compiler_params=pltpu.CompilerParams(needs_layout_passes=False),
        out_type=jax.ShapeDtypeStruct((_NW, _HPAD), jnp.int32),
        scratch_types=[
            pltpu.VMEM((_HPAD,), jnp.int32),
            pltpu.VMEM((_CH_ROWS, _COLS), jnp.int32),
            pltpu.VMEM((_CH_ROWS, _COLS), jnp.int32),
            pltpu.SemaphoreType.DMA,
            pltpu.SemaphoreType.DMA,
        ],
    )


_BLKW = 100096
_NBLK = _HPAD // _BLKW


def _tc_entropy(part_ref, out_ref, acc_ref):
    i = pl.program_id(0)

    @pl.when(i == 0)
    def _():
        acc_ref[0] = 0.0
        acc_ref[1] = 0.0

    c = jnp.sum(part_ref[...], axis=0, keepdims=True).astype(jnp.float32)
    mask = c > 0.0
    safe = jnp.where(mask, c, 1.0)
    acc_ref[0] += jnp.sum(safe * jnp.log(safe))
    acc_ref[1] += jnp.sum(mask.astype(jnp.float32))

    @pl.when(i == _NBLK - 1)
    def _():
        s = acc_ref[0]
        nnz = acc_ref[1]
        logn = np.float32(np.log(float(_N)))
        actual = logn - s / np.float32(_N)
        out_ref[0, 0] = actual / jnp.log(nnz)


_tc_call = pl.pallas_call(
    _tc_entropy,
    grid=(_NBLK,),
    in_specs=[pl.BlockSpec((_NW, _BLKW), lambda i: (0, i))],
    out_specs=pl.BlockSpec(memory_space=pltpu.SMEM),
    out_shape=jax.ShapeDtypeStruct((1, 1), jnp.float32),
    scratch_shapes=[pltpu.SMEM((2,), jnp.float32)],
)


def kernel(recommendations):
    partials = _get_sc_hist()(recommendations)
    out = _tc_call(partials)
    return out[0, 0]


# speedup vs baseline: 1.0264x; 1.0264x over previous
import functools

import numpy as np
import jax
import jax.numpy as jnp
from jax import lax
from jax.experimental import pallas as pl
from jax.experimental.pallas import tpu as pltpu
from jax.experimental.pallas import tpu_sc as plsc

_VOCAB = 100000
_HPAD = 100096
_NC, _NS, _L = 2, 16, 16
_NW = _NC * _NS
_ROWS, _COLS = 16384, 200
_N = _ROWS * _COLS
_PER_W = _N // _NW
_CHUNK = 6400
_NCHUNK = _PER_W // _CHUNK
_GROUPS = _CHUNK // _L
_RUNROLL = 1

_CH_ROWS = _CHUNK // _COLS
_ROWS_W = _ROWS // _NW


def _sc_hist_body(rec_hbm, out_hbm, hist, buf0, buf1, sem0, sem1):
    wid = lax.axis_index("s") * _NC + lax.axis_index("c")
    base_row = wid * _ROWS_W
    bufs, sems = (buf0, buf1), (sem0, sem1)

    pending = pltpu.async_copy(
        rec_hbm.at[pl.ds(base_row, _CH_ROWS), :], bufs[0], sems[0]
    )

    zeros = jnp.zeros((_L,), jnp.int32)

    def _zero(i, carry):
        hist[pl.ds(i * _L, _L)] = zeros
        return carry

    lax.fori_loop(0, _HPAD // _L, _zero, 0, unroll=8)

    ones = jnp.ones((_L,), jnp.int32)
    _himask = jnp.arange(_L, dtype=jnp.int32) >= _L - _COLS % _L
    for c in range(_NCHUNK):
        cur = pending
        if c + 1 < _NCHUNK:
            pending = pltpu.async_copy(
                rec_hbm.at[pl.ds(base_row + (c + 1) * _CH_ROWS, _CH_ROWS), :],
                bufs[(c + 1) % 2],
                sems[(c + 1) % 2],
            )
        cur.wait()
        buf = bufs[c % 2]

        def _rows(r0, carry2):
            for r in range(_RUNROLL):
                idxs = [buf[r0 + r, pl.ds(g * _L, _L)] for g in range(12)]
                tail = buf[r0 + r, pl.ds(_COLS - _L, _L)]
                for idx in idxs:
                    plsc.addupdate_scatter(hist, [idx], ones)
                plsc.addupdate_scatter(hist, [tail], ones, mask=_himask)
            return carry2

        lax.fori_loop(0, _CH_ROWS // _RUNROLL,
                      lambda r, c2: _rows(r * _RUNROLL, c2), 0)

    pltpu.sync_copy(hist, out_hbm.at[wid])


@functools.cache
def _get_sc_hist():
    mesh = plsc.VectorSubcoreMesh(
        core_axis_name="c", subcore_axis_name="s", num_cores=_NC, num_subcores=_NS
    )
    return pl.kernel(
        _sc_hist_body,
        mesh=mesh,
        compiler_params=pltpu.CompilerParams(needs_layout_passes=False),
        out_type=jax.ShapeDtypeStruct((_NW, _HPAD), jnp.int32),
        scratch_types=[
            pltpu.VMEM((_HPAD,), jnp.int32),
            pltpu.VMEM((_CH_ROWS, _COLS), jnp.int32),
            pltpu.VMEM((_CH_ROWS, _COLS), jnp.int32),
            pltpu.SemaphoreType.DMA,
            pltpu.SemaphoreType.DMA,
        ],
    )


_BLKW = 50048
_NBLK = _HPAD // _BLKW


def _tc_entropy(part_ref, out_ref, acc_ref):
    i = pl.program_id(0)

    @pl.when(i == 0)
    def _():
        acc_ref[0] = 0.0
        acc_ref[1] = 0.0

    c = jnp.sum(part_ref[...], axis=0, keepdims=True).astype(jnp.float32)
    mask = c > 0.0
    safe = jnp.where(mask, c, 1.0)
    acc_ref[0] += jnp.sum(safe * jnp.log(safe))
    acc_ref[1] += jnp.sum(mask.astype(jnp.float32))

    @pl.when(i == _NBLK - 1)
    def _():
        s = acc_ref[0]
        nnz = acc_ref[1]
        logn = np.float32(np.log(float(_N)))
        actual = logn - s / np.float32(_N)
        out_ref[0, 0] = actual / jnp.log(nnz)


_tc_call = pl.pallas_call(
    _tc_entropy,
    grid=(_NBLK,),
    in_specs=[pl.BlockSpec((_NW, _BLKW), lambda i: (0, i))],
    out_specs=pl.BlockSpec(memory_space=pltpu.SMEM),
    out_shape=jax.ShapeDtypeStruct((1, 1), jnp.float32),
    scratch_shapes=[pltpu.SMEM((2,), jnp.float32)],
)


def kernel(recommendations):
    partials = _get_sc_hist()(recommendations)
    out = _tc_call(partials)
    return out[0, 0]
